# baseline (speedup 1.0000x reference)
import functools

import jax
import jax.numpy as jnp
from jax import lax
from jax.experimental import pallas as pl
from jax.experimental.pallas import tpu as pltpu
from jax.experimental.pallas import tpu_sc as plsc

HIDDEN = 3072
SEQ = 4096
NUM_TOKENS = 2 * SEQ
NC = 2
NS = 16
NW = NC * NS
PER_W = NUM_TOKENS // NW
CHUNK = 16
NCHUNK = PER_W // CHUNK
NBUF = 2
DMA_ROWS = 64
DMA_CHUNKS = DMA_ROWS // CHUNK

_mesh = plsc.VectorSubcoreMesh(core_axis_name="c", subcore_axis_name="s")


@functools.partial(
    pl.kernel,
    mesh=_mesh,
    out_type=jax.ShapeDtypeStruct((NUM_TOKENS, HIDDEN), jnp.float32),
    scratch_types=[
        pltpu.VMEM((PER_W,), jnp.int32),
        pltpu.VMEM((NBUF, CHUNK, HIDDEN), jnp.float32),
        pltpu.SemaphoreType.DMA,
        pltpu.SemaphoreType.DMA,
        pltpu.SemaphoreType.DMA,
        pltpu.SemaphoreType.DMA,
        pltpu.SemaphoreType.DMA,
    ],
)
def _embed_lookup(
    table_hbm, ids_hbm, out_hbm, idx_v, rows_v, si0, si1, so0, so1, dsem
):
    in_sem = (si0, si1)
    out_sem = (so0, so1)
    wid = lax.axis_index("s") * NC + lax.axis_index("c")
    base = wid * PER_W
    tiles_per_row = SEQ // PER_W
    ids_src = ids_hbm.at[
        wid // tiles_per_row, pl.ds((wid % tiles_per_row) * PER_W, PER_W)
    ]
    pltpu.sync_copy(ids_src, idx_v)

    def fire_group(g, _):
        vec = idx_v[pl.ds(g * 16, 16)]
        for k in range(16):
            pltpu.async_copy(
                table_hbm.at[pl.ds(vec[k], 1)],
                out_hbm.at[pl.ds(base + g * 16 + k, 1)],
                dsem,
            )
        return _

    lax.fori_loop(0, DMA_ROWS // 16, fire_group, 0)

    def gather(j, b):
        return pltpu.async_copy(
            table_hbm.at[idx_v.at[pl.ds(j * CHUNK, CHUNK)]], rows_v.at[b], in_sem[b]
        )

    def put(j, b):
        return pltpu.async_copy(
            rows_v.at[b], out_hbm.at[pl.ds(base + j * CHUNK, CHUNK)], out_sem[b]
        )

    gcp = [gather(DMA_CHUNKS, 0), gather(DMA_CHUNKS + 1, 1)]
    pcp = [None, None]
    for j in range(DMA_CHUNKS, NCHUNK):
        b = j % NBUF
        gcp[b].wait()
        pcp[b] = put(j, b)
        if j + NBUF < NCHUNK:
            pcp[b].wait()
            gcp[b] = gather(j + NBUF, b)
    pcp[0].wait()
    pcp[1].wait()

    def drain_row(i, _):
        pltpu.make_async_copy(
            table_hbm.at[pl.ds(0, 1)], out_hbm.at[pl.ds(base, 1)], dsem
        ).wait()
        return _

    lax.fori_loop(0, DMA_ROWS, drain_row, 0)


def kernel(embed_weight, input_ids):
    batch, seq_len = input_ids.shape
    out = _embed_lookup(embed_weight, input_ids.astype(jnp.int32))
    return out.reshape(batch, seq_len, HIDDEN)


# speedup vs baseline: 8.6683x; 8.6683x over previous
import functools

import jax
import jax.numpy as jnp
from jax import lax
from jax.experimental import pallas as pl
from jax.experimental.pallas import tpu as pltpu
from jax.experimental.pallas import tpu_sc as plsc

HIDDEN = 3072
SEQ = 4096
NUM_TOKENS = 2 * SEQ
NC = 2
NS = 16
NW = NC * NS
PER_W = NUM_TOKENS // NW
CHUNK = 16
NCHUNK = PER_W // CHUNK
NBUF = 2

_mesh = plsc.VectorSubcoreMesh(core_axis_name="c", subcore_axis_name="s")


@functools.partial(
    pl.kernel,
    mesh=_mesh,
    out_type=jax.ShapeDtypeStruct((2, SEQ, HIDDEN), jnp.float32),
    scratch_types=[
        pltpu.VMEM((PER_W,), jnp.int32),
        pltpu.VMEM((NBUF, CHUNK, HIDDEN), jnp.float32),
        pltpu.SemaphoreType.DMA,
        pltpu.SemaphoreType.DMA,
        pltpu.SemaphoreType.DMA,
        pltpu.SemaphoreType.DMA,
    ],
)
def _embed_lookup(table_hbm, ids_hbm, out_hbm, idx_v, rows_v, si0, si1, so0, so1):
    in_sem = (si0, si1)
    out_sem = (so0, so1)
    wid = lax.axis_index("s") * NC + lax.axis_index("c")
    tiles_per_row = SEQ // PER_W
    brow = wid // tiles_per_row
    bcol = (wid % tiles_per_row) * PER_W
    pltpu.sync_copy(ids_hbm.at[brow, pl.ds(bcol, PER_W)], idx_v)

    def gather(j, b):
        return pltpu.async_copy(
            table_hbm.at[idx_v.at[pl.ds(j * CHUNK, CHUNK)]], rows_v.at[b], in_sem[b]
        )

    def put(j, b):
        return pltpu.async_copy(
            rows_v.at[b], out_hbm.at[brow, pl.ds(bcol + j * CHUNK, CHUNK)], out_sem[b]
        )

    gcp = [gather(0, 0), gather(1, 1)]
    pcp = [None, None]
    for j in range(NCHUNK):
        b = j % NBUF
        gcp[b].wait()
        pcp[b] = put(j, b)
        if j + NBUF < NCHUNK:
            pcp[b].wait()
            gcp[b] = gather(j + NBUF, b)
    pcp[0].wait()
    pcp[1].wait()


def kernel(embed_weight, input_ids):
    return _embed_lookup(embed_weight, input_ids.astype(jnp.int32))
